# baseline (speedup 1.0000x reference)
import functools

import jax
import jax.numpy as jnp
from jax import lax
from jax.experimental import pallas as pl
from jax.experimental.pallas import tpu as pltpu
from jax.experimental.pallas import tpu_sc as plsc

_N = 10000
_E = 320000
_D = 128
_NC = 2
_NS = 16
_NW = _NC * _NS
_EPT = _E // _NW
_K = 125
_NCHUNK = _EPT // _K
_NBUF = 4
_RPT = 624
_RREM = _N - _NS * _RPT
_WBC = 104

_mesh = plsc.VectorSubcoreMesh(core_axis_name="c", subcore_axis_name="s")


@functools.partial(
    pl.kernel,
    out_type=jax.ShapeDtypeStruct((_NC, _N, _D), jnp.float32),
    mesh=_mesh,
    compiler_params=pltpu.CompilerParams(
        use_tc_tiling_on_sc=False, needs_layout_passes=False
    ),
    scratch_types=[
        pltpu.VMEM((_NCHUNK, _K), jnp.int32),
        pltpu.VMEM((1, _K), jnp.int32),
        pltpu.VMEM((1, _K), jnp.int32),
        pltpu.VMEM((1, _K), jnp.int32),
        pltpu.VMEM((1, _K), jnp.int32),
        pltpu.VMEM((_K, _D), jnp.bfloat16),
        pltpu.VMEM((_K, _D), jnp.bfloat16),
        pltpu.VMEM((_K, _D), jnp.bfloat16),
        pltpu.VMEM((_K, _D), jnp.bfloat16),
        pltpu.VMEM_SHARED((_N, _D), jnp.bfloat16),
        pltpu.VMEM((_WBC, _D), jnp.bfloat16),
        pltpu.VMEM((_WBC, _D), jnp.float32),
        pltpu.SemaphoreType.DMA,
        pltpu.SemaphoreType.DMA,
        pltpu.SemaphoreType.DMA,
        pltpu.SemaphoreType.DMA,
        pltpu.SemaphoreType.DMA,
        pltpu.SemaphoreType.DMA,
        pltpu.SemaphoreType.DMA,
        pltpu.SemaphoreType.DMA,
        pltpu.SemaphoreType.DMA,
        pltpu.SemaphoreType.DMA,
        pltpu.SemaphoreType.DMA,
        pltpu.SemaphoreType.DMA,
    ],
)
def _sc_agg(x_hbm, src_hbm, dst_hbm, out_hbm,
            src_v, dst0_v, dst1_v, dst2_v, dst3_v,
            rows0_v, rows1_v, rows2_v, rows3_v, acc_sh, wb_bf, wb_f32,
            gsem0, gsem1, gsem2, gsem3, dsem0, dsem1, dsem2, dsem3,
            ssem0, ssem1, ssem2, ssem3):
    c = lax.axis_index("c")
    s = lax.axis_index("s")
    wid = c * _NS + s
    r0 = s * _RPT
    pltpu.sync_copy(x_hbm.at[pl.ds(r0, _RPT)], acc_sh.at[pl.ds(r0, _RPT)])

    @pl.when(s == _NS - 1)
    def _init_tail():
        rt = _NS * _RPT
        pltpu.sync_copy(x_hbm.at[pl.ds(rt, _RREM)], acc_sh.at[pl.ds(rt, _RREM)])

    plsc.subcore_barrier()

    pltpu.sync_copy(src_hbm.at[wid], src_v)

    gsems = (gsem0, gsem1, gsem2, gsem3)
    dsems = (dsem0, dsem1, dsem2, dsem3)
    ssems = (ssem0, ssem1, ssem2, ssem3)
    dsts = (dst0_v, dst1_v, dst2_v, dst3_v)
    rows = (rows0_v, rows1_v, rows2_v, rows3_v)

    def _prefetch(i, b):
        pltpu.async_copy(dst_hbm.at[wid, pl.ds(i, 1)], dsts[b], dsems[b])
        pltpu.async_copy(x_hbm.at[src_v.at[i]], rows[b], gsems[b])

    def _drain_scatter(b):
        pltpu.make_async_copy(
            rows[b], acc_sh.at[dsts[b].at[0]], ssems[b]
        ).wait()

    def _step(i, b, drain_prev, prefetch_next):
        pltpu.make_async_copy(dst_hbm.at[wid, pl.ds(i, 1)], dsts[b], dsems[b]).wait()
        pltpu.make_async_copy(x_hbm.at[src_v.at[i]], rows[b], gsems[b]).wait()
        pltpu.async_copy(rows[b], acc_sh.at[dsts[b].at[0]], ssems[b], add=True)
        b2 = (b + _NBUF - 1) % _NBUF
        if drain_prev:
            _drain_scatter(b2)
        if prefetch_next:
            _prefetch(i + _NBUF - 1, b2)

    for b in range(_NBUF - 1):
        _prefetch(b, b)
    _step(0, 0, drain_prev=False, prefetch_next=True)

    @pl.loop(1, _NCHUNK - 7, step=_NBUF)
    def _chunk(g):
        for k in range(_NBUF):
            _step(g + k, (1 + k) % _NBUF, drain_prev=True, prefetch_next=True)

    for i in range(_NCHUNK - 7, _NCHUNK):
        _step(i, i % _NBUF, drain_prev=True,
              prefetch_next=(i + _NBUF - 1 < _NCHUNK))
    _drain_scatter((_NCHUNK - 1) % _NBUF)
    plsc.subcore_barrier()

    def _wb_rows(nr, rbase):
        pltpu.sync_copy(acc_sh.at[pl.ds(rbase, nr)], wb_bf.at[pl.ds(0, nr)])

        @pl.loop(0, nr)
        def _r(q):
            for g in range(4):
                v = wb_bf[q, pl.ds(32 * g, 32)]
                ev, od = plsc.unpack(v, format=plsc.PackFormat.INTERLEAVED)
                wb_f32[q, pl.ds(32 * g, 16)] = ev
                wb_f32[q, pl.ds(32 * g + 16, 16)] = od

        pltpu.sync_copy(wb_f32.at[pl.ds(0, nr)], out_hbm.at[c, pl.ds(rbase, nr)])

    for cc in range(_RPT // _WBC):
        _wb_rows(_WBC, r0 + cc * _WBC)

    @pl.when(s == _NS - 1)
    def _wb_tail():
        _wb_rows(_RREM, _NS * _RPT)


def _mlp_body(x_ref, agg_ref, ws_ref, w_ref, b_ref, out_ref):
    aggs = agg_ref[0] + agg_ref[1]
    out_ref[...] = (
        jnp.dot(aggs, ws_ref[...], preferred_element_type=jnp.float32)
        - jnp.dot(x_ref[...], w_ref[...], preferred_element_type=jnp.float32)
        + b_ref[...]
    )


_RB = 1000

_mlp = pl.pallas_call(
    _mlp_body,
    grid=(_N // _RB,),
    in_specs=[
        pl.BlockSpec((_RB, _D), lambda i: (i, 0)),
        pl.BlockSpec((_NC, _RB, _D), lambda i: (0, i, 0)),
        pl.BlockSpec((_D, _D), lambda i: (0, 0)),
        pl.BlockSpec((_D, _D), lambda i: (0, 0)),
        pl.BlockSpec((1, _D), lambda i: (0, 0)),
    ],
    out_specs=pl.BlockSpec((_RB, _D), lambda i: (i, 0)),
    out_shape=jax.ShapeDtypeStruct((_N, _D), jnp.float32),
)

_PERM = [
    32 * g + (2 * k if k < 16 else 2 * (k - 16) + 1)
    for g in range(4)
    for k in range(32)
]


def kernel(x, edge_index, W, b):
    src = edge_index[0].reshape(_NW, _NCHUNK, _K)
    dst = edge_index[1].reshape(_NW, _NCHUNK, _K)
    agg2 = _sc_agg(x.astype(jnp.bfloat16), src, dst)
    Ws = W[jnp.asarray(_PERM, dtype=jnp.int32), :]
    return _mlp(x, agg2, Ws, W, b.reshape(1, _D))


# speedup vs baseline: 1.1299x; 1.1299x over previous
import functools

import jax
import jax.numpy as jnp
from jax import lax
from jax.experimental import pallas as pl
from jax.experimental.pallas import tpu as pltpu
from jax.experimental.pallas import tpu_sc as plsc

_N = 10000
_E = 320000
_D = 128
_NC = 2
_NS = 16
_NW = _NC * _NS
_EPT = _E // _NW
_K = 80
_NCHUNK = _EPT // _K
_NBUF = 3
_RPT = 624
_RREM = _N - _NS * _RPT
_WBC = 104

_mesh = plsc.VectorSubcoreMesh(core_axis_name="c", subcore_axis_name="s")


@functools.partial(
    pl.kernel,
    out_type=jax.ShapeDtypeStruct((_NC, _N, _D), jnp.float32),
    mesh=_mesh,
    compiler_params=pltpu.CompilerParams(
        use_tc_tiling_on_sc=False, needs_layout_passes=False
    ),
    scratch_types=[
        pltpu.VMEM((_EPT,), jnp.int32),
        pltpu.VMEM((_EPT,), jnp.int32),
        pltpu.VMEM((_K, _D), jnp.bfloat16),
        pltpu.VMEM((_K, _D), jnp.bfloat16),
        pltpu.VMEM((_K, _D), jnp.bfloat16),
        pltpu.VMEM_SHARED((_N, _D), jnp.bfloat16),
        pltpu.VMEM((_WBC, _D), jnp.bfloat16),
        pltpu.VMEM((_WBC, _D), jnp.float32),
        pltpu.SemaphoreType.DMA,
        pltpu.SemaphoreType.DMA,
        pltpu.SemaphoreType.DMA,
        pltpu.SemaphoreType.DMA,
        pltpu.SemaphoreType.DMA,
        pltpu.SemaphoreType.DMA,
    ],
)
def _sc_agg(x_hbm, ei_hbm, out_hbm,
            src_v, dst_v, rows0_v, rows1_v, rows2_v, acc_sh, wb_bf, wb_f32,
            gsem0, gsem1, gsem2, ssem0, ssem1, ssem2):
    c = lax.axis_index("c")
    s = lax.axis_index("s")
    wid = c * _NS + s
    r0 = s * _RPT
    pltpu.sync_copy(x_hbm.at[pl.ds(r0, _RPT)], acc_sh.at[pl.ds(r0, _RPT)])

    @pl.when(s == _NS - 1)
    def _init_tail():
        rt = _NS * _RPT
        pltpu.sync_copy(x_hbm.at[pl.ds(rt, _RREM)], acc_sh.at[pl.ds(rt, _RREM)])

    plsc.subcore_barrier()

    ebase = wid * _EPT
    pltpu.sync_copy(ei_hbm.at[0, pl.ds(ebase, _EPT)], src_v)
    pltpu.sync_copy(ei_hbm.at[1, pl.ds(ebase, _EPT)], dst_v)

    gsems = (gsem0, gsem1, gsem2)
    ssems = (ssem0, ssem1, ssem2)
    rows = (rows0_v, rows1_v, rows2_v)

    def _prefetch(i, b):
        pltpu.async_copy(
            x_hbm.at[src_v.at[pl.ds(i * _K, _K)]], rows[b], gsems[b]
        )

    def _drain_scatter(i, b):
        pltpu.make_async_copy(
            rows[b], acc_sh.at[dst_v.at[pl.ds(i * _K, _K)]], ssems[b]
        ).wait()

    def _step(i, b, drain_prev, prefetch_next):
        pltpu.make_async_copy(
            x_hbm.at[src_v.at[pl.ds(i * _K, _K)]], rows[b], gsems[b]
        ).wait()
        pltpu.async_copy(
            rows[b], acc_sh.at[dst_v.at[pl.ds(i * _K, _K)]], ssems[b], add=True
        )
        b2 = (b + _NBUF - 1) % _NBUF
        if drain_prev:
            _drain_scatter(i - 1, b2)
        if prefetch_next:
            _prefetch(i + _NBUF - 1, b2)

    for b in range(_NBUF - 1):
        _prefetch(b, b)
    _step(0, 0, drain_prev=False, prefetch_next=True)

    @pl.loop(1, _NCHUNK - 4, step=_NBUF)
    def _chunk(g):
        for k in range(_NBUF):
            _step(g + k, (1 + k) % _NBUF, drain_prev=True, prefetch_next=True)

    for i in range(_NCHUNK - 4, _NCHUNK):
        _step(i, i % _NBUF, drain_prev=True,
              prefetch_next=(i + _NBUF - 1 < _NCHUNK))
    _drain_scatter(_NCHUNK - 1, (_NCHUNK - 1) % _NBUF)
    plsc.subcore_barrier()

    def _wb_rows(nr, rbase):
        pltpu.sync_copy(acc_sh.at[pl.ds(rbase, nr)], wb_bf.at[pl.ds(0, nr)])

        @pl.loop(0, nr)
        def _r(q):
            for g in range(4):
                v = wb_bf[q, pl.ds(32 * g, 32)]
                ev, od = plsc.unpack(v, format=plsc.PackFormat.INTERLEAVED)
                wb_f32[q, pl.ds(32 * g, 16)] = ev
                wb_f32[q, pl.ds(32 * g + 16, 16)] = od

        pltpu.sync_copy(wb_f32.at[pl.ds(0, nr)], out_hbm.at[c, pl.ds(rbase, nr)])

    for cc in range(_RPT // _WBC):
        _wb_rows(_WBC, r0 + cc * _WBC)

    @pl.when(s == _NS - 1)
    def _wb_tail():
        _wb_rows(_RREM, _NS * _RPT)


def _mlp_body(x_ref, agg_ref, ws_ref, w_ref, b_ref, out_ref):
    aggs = agg_ref[0] + agg_ref[1]
    out_ref[...] = (
        jnp.dot(aggs, ws_ref[...], preferred_element_type=jnp.float32)
        - jnp.dot(x_ref[...], w_ref[...], preferred_element_type=jnp.float32)
        + b_ref[...]
    )


_RB = 1000

_mlp = pl.pallas_call(
    _mlp_body,
    grid=(_N // _RB,),
    in_specs=[
        pl.BlockSpec((_RB, _D), lambda i: (i, 0)),
        pl.BlockSpec((_NC, _RB, _D), lambda i: (0, i, 0)),
        pl.BlockSpec((_D, _D), lambda i: (0, 0)),
        pl.BlockSpec((_D, _D), lambda i: (0, 0)),
        pl.BlockSpec((1, _D), lambda i: (0, 0)),
    ],
    out_specs=pl.BlockSpec((_RB, _D), lambda i: (i, 0)),
    out_shape=jax.ShapeDtypeStruct((_N, _D), jnp.float32),
)

_PERM = [
    32 * g + (2 * k if k < 16 else 2 * (k - 16) + 1)
    for g in range(4)
    for k in range(32)
]


def kernel(x, edge_index, W, b):
    agg2 = _sc_agg(x.astype(jnp.bfloat16), edge_index)
    Ws = W[jnp.asarray(_PERM, dtype=jnp.int32), :]
    return _mlp(x, agg2, Ws, W, b.reshape(1, _D))
